# baseline (speedup 1.0000x reference)
import jax
import jax.numpy as jnp
from jax.experimental import pallas as pl

_DIM = 768
_P = 8
_N = 1024
_H = 32


def _dsaam_kernel(x_ref, rp_ref, Wv_ref, bv_ref, Wcat_ref, bcat_ref, Wo_ref,
                  bo_ref, out_ref):
  for s in range(2):
      x = x_ref[s]

      value = jnp.dot(x, Wv_ref[...], preferred_element_type=jnp.float32)
      value = value + bv_ref[...]

      cat = jnp.dot(x, Wcat_ref[...], preferred_element_type=jnp.float32)
      cat = cat + bcat_ref[...]
      catT = cat.T
      offxT = catT[0:_P]
      offyT = catT[_P:2 * _P]
      awlT = catT[2 * _P:3 * _P]

      m = jnp.max(awlT, axis=0, keepdims=True)
      e = jnp.exp(awlT - m)
      awT = e / jnp.sum(e, axis=0, keepdims=True)

      rpT = rp_ref[s].T
      scale = (_H - 1) * 0.5
      gxT = (jnp.clip(rpT[0:1] + offxT, -1.0, 1.0) + 1.0) * scale
      gyT = (jnp.clip(rpT[1:2] + offyT, -1.0, 1.0) + 1.0) * scale

      grow = jax.lax.broadcasted_iota(jnp.int32, (_H, 1), 0).astype(jnp.float32)

      St3 = jnp.zeros((_H, _H, _N), jnp.bfloat16)
      for p in range(_P):
          Xp = jnp.maximum(0.0, 1.0 - jnp.abs(gxT[p:p + 1] - grow))
          Yp = jnp.maximum(0.0, 1.0 - jnp.abs(gyT[p:p + 1] - grow))
          Yp = awT[p:p + 1] * Yp
          St3 = St3 + Yp.astype(jnp.bfloat16)[:, None, :] * \
              Xp.astype(jnp.bfloat16)[None, :, :]
      St = St3.reshape(_N, _N)

      sampled = jax.lax.dot_general(St, value.astype(jnp.bfloat16),
                                    (((0,), (0,)), ((), ())),
                                    preferred_element_type=jnp.float32)
      out = jnp.dot(sampled, Wo_ref[...], preferred_element_type=jnp.float32)
      out_ref[s] = out + bo_ref[...]


def kernel(x, ref_points, Wv, bv, Woff, boff, Waw, baw, Wo, bo):
    B, N, C = x.shape
    Woff3 = Woff.reshape(C, _P, 2)
    Wcat = jnp.concatenate([Woff3[:, :, 0], Woff3[:, :, 1], Waw], axis=1)
    boff3 = boff.reshape(_P, 2)
    bcat = jnp.concatenate([boff3[:, 0], boff3[:, 1], baw]).reshape(1, 3 * _P)

    grid = (B // 2,)
    out = pl.pallas_call(
        _dsaam_kernel,
        grid=grid,
        in_specs=[
            pl.BlockSpec((2, N, C), lambda b: (b, 0, 0)),
            pl.BlockSpec((2, N, 2), lambda b: (b, 0, 0)),
            pl.BlockSpec((C, C), lambda b: (0, 0)),
            pl.BlockSpec((1, C), lambda b: (0, 0)),
            pl.BlockSpec((C, 3 * _P), lambda b: (0, 0)),
            pl.BlockSpec((1, 3 * _P), lambda b: (0, 0)),
            pl.BlockSpec((C, C), lambda b: (0, 0)),
            pl.BlockSpec((1, C), lambda b: (0, 0)),
        ],
        out_specs=pl.BlockSpec((2, N, C), lambda b: (b, 0, 0)),
        out_shape=jax.ShapeDtypeStruct((B, N, C), jnp.float32),
    )(x, ref_points, Wv, bv.reshape(1, C), Wcat, bcat, Wo, bo.reshape(1, C))
    return out


# speedup vs baseline: 1.0003x; 1.0003x over previous
import jax
import jax.numpy as jnp
from jax.experimental import pallas as pl

_DIM = 768
_P = 8
_N = 1024
_H = 32


def _dsaam_kernel(x_ref, rp_ref, Wv_ref, bv_ref, Wcat_ref, bcat_ref, Wo_ref,
                  bo_ref, out_ref):
  for s in range(2):
      x = x_ref[s]

      value = jnp.dot(x, Wv_ref[...], preferred_element_type=jnp.float32)
      value = value + bv_ref[...]

      catT = jax.lax.dot_general(Wcat_ref[...], x, (((0,), (1,)), ((), ())),
                                 preferred_element_type=jnp.float32)
      catT = catT + bcat_ref[...]
      offxT = catT[0:_P]
      offyT = catT[_P:2 * _P]
      awlT = catT[2 * _P:3 * _P]

      m = jnp.max(awlT, axis=0, keepdims=True)
      e = jnp.exp(awlT - m)
      awT = e / jnp.sum(e, axis=0, keepdims=True)

      rpT = rp_ref[s].T
      scale = (_H - 1) * 0.5
      gxT = (jnp.clip(rpT[0:1] + offxT, -1.0, 1.0) + 1.0) * scale
      gyT = (jnp.clip(rpT[1:2] + offyT, -1.0, 1.0) + 1.0) * scale

      grow = jax.lax.broadcasted_iota(jnp.int32, (_H, 1), 0).astype(jnp.float32)

      St3 = jnp.zeros((_H, _H, _N), jnp.bfloat16)
      for p in range(_P):
          Xp = jnp.maximum(0.0, 1.0 - jnp.abs(gxT[p:p + 1] - grow))
          Yp = jnp.maximum(0.0, 1.0 - jnp.abs(gyT[p:p + 1] - grow))
          Yp = awT[p:p + 1] * Yp
          St3 = St3 + Yp.astype(jnp.bfloat16)[:, None, :] * \
              Xp.astype(jnp.bfloat16)[None, :, :]
      St = St3.reshape(_N, _N)

      sampled = jax.lax.dot_general(St, value.astype(jnp.bfloat16),
                                    (((0,), (0,)), ((), ())),
                                    preferred_element_type=jnp.float32)
      out = jnp.dot(sampled, Wo_ref[...], preferred_element_type=jnp.float32)
      out_ref[s] = out + bo_ref[...]


def kernel(x, ref_points, Wv, bv, Woff, boff, Waw, baw, Wo, bo):
    B, N, C = x.shape
    Woff3 = Woff.reshape(C, _P, 2)
    Wcat = jnp.concatenate([Woff3[:, :, 0], Woff3[:, :, 1], Waw], axis=1)
    boff3 = boff.reshape(_P, 2)
    bcat = jnp.concatenate([boff3[:, 0], boff3[:, 1], baw]).reshape(3 * _P, 1)

    grid = (B // 2,)
    out = pl.pallas_call(
        _dsaam_kernel,
        grid=grid,
        in_specs=[
            pl.BlockSpec((2, N, C), lambda b: (b, 0, 0)),
            pl.BlockSpec((2, N, 2), lambda b: (b, 0, 0)),
            pl.BlockSpec((C, C), lambda b: (0, 0)),
            pl.BlockSpec((1, C), lambda b: (0, 0)),
            pl.BlockSpec((C, 3 * _P), lambda b: (0, 0)),
            pl.BlockSpec((3 * _P, 1), lambda b: (0, 0)),
            pl.BlockSpec((C, C), lambda b: (0, 0)),
            pl.BlockSpec((1, C), lambda b: (0, 0)),
        ],
        out_specs=pl.BlockSpec((2, N, C), lambda b: (b, 0, 0)),
        out_shape=jax.ShapeDtypeStruct((B, N, C), jnp.float32),
    )(x, ref_points, Wv, bv.reshape(1, C), Wcat, bcat, Wo, bo.reshape(1, C))
    return out
